# baseline (speedup 1.0000x reference)
import functools

import jax
import jax.numpy as jnp
from jax import lax
from jax.experimental import pallas as pl
from jax.experimental.pallas import tpu as pltpu
from jax.experimental.pallas import tpu_sc as plsc

N = 10000
E = 320000
D = 128
C = 25

NC = 2
NS = 16
NW = NC * NS
CH = 128
NCH = 80
EPW = CH * NCH
EPAD = EPW * NW
NP = 10240
RPT = NP // NS

RB = 512
G = 20


def _sc_mesh():
    return plsc.VectorSubcoreMesh(core_axis_name="c", subcore_axis_name="s")


def _hist_body(dst_hbm, out_hbm, didxs, ones_v, zb, hbuf, ssems, hist_sh):
    c = lax.axis_index("c")
    s = lax.axis_index("s")
    tb = s * RPT
    for i in range(CH // 16):
        ones_v[pl.ds(i * 16, 16)] = jnp.ones((16,), jnp.float32)
        zb[pl.ds(i * 16, 16)] = jnp.zeros((16,), jnp.float32)
    for k in range(RPT // CH):
        pltpu.sync_copy(zb, hist_sh.at[pl.ds(tb + k * CH, CH)])
    plsc.subcore_barrier()
    w = c * NS + s

    def fetch(chunk, b):
        pltpu.sync_copy(dst_hbm.at[pl.ds((w * NCH + chunk) * CH, CH)],
                        didxs[b])

    def scat(b):
        pltpu.make_async_copy(ones_v, hist_sh.at[didxs[b]], ssems[b]
                              ).start(add=True)

    def scat_wait(b):
        pltpu.make_async_copy(ones_v, hist_sh.at[didxs[b]], ssems[b]).wait()

    for b in range(2):
        fetch(b, b)
        scat(b)

    def body(jj, carry):
        for b in range(2):
            cb = jj * 2 + b

            @pl.when(jj < NCH // 2 - 1)
            def _():
                scat_wait(b)
                fetch(cb + 2, b)
                scat(b)

        return carry

    lax.fori_loop(0, NCH // 2, body, 0)
    for b in range(2):
        scat_wait(b)
    plsc.subcore_barrier()
    pltpu.sync_copy(hist_sh.at[pl.ds(tb, RPT)], hbuf)
    pltpu.sync_copy(hbuf, out_hbm.at[c, pl.ds(tb, RPT)])


def _sc_hist(dst_pad):
    k = functools.partial(
        pl.kernel,
        out_type=jax.ShapeDtypeStruct((NC, NP), jnp.float32),
        mesh=_sc_mesh(),
        scratch_types=[
            [pltpu.VMEM((CH,), jnp.int32) for _ in range(2)],
            pltpu.VMEM((CH,), jnp.float32),
            pltpu.VMEM((CH,), jnp.float32),
            pltpu.VMEM((RPT,), jnp.float32),
            [pltpu.SemaphoreType.DMA for _ in range(2)],
            pltpu.VMEM_SHARED((NP,), jnp.float32),
        ],
    )(_hist_body)
    return k(dst_pad)


def _scatter_body(y_hbm, src_hbm, dst_hbm, z_hbm, out_hbm,
                  sbufs, dbufs, rows, sem, acc_sh):
    c = lax.axis_index("c")
    s = lax.axis_index("s")
    tb = s * RPT
    pltpu.sync_copy(z_hbm, rows)
    for k in range(RPT // CH):
        pltpu.sync_copy(rows, acc_sh.at[pl.ds(tb + k * CH, CH)])
    plsc.subcore_barrier()
    w = c * NS + s

    def fetch(chunk, b):
        base = (w * NCH + chunk) * CH
        pltpu.sync_copy(src_hbm.at[pl.ds(base, CH)], sbufs[b])
        pltpu.sync_copy(dst_hbm.at[pl.ds(base, CH)], dbufs[b])

    fetch(0, 0)

    def body(jj, carry):
        for b in range(2):
            cb = jj * 2 + b
            g = pltpu.make_async_copy(y_hbm.at[sbufs[b]], rows, sem)
            g.start()

            @pl.when(cb < NCH - 1)
            def _():
                fetch(cb + 1, 1 - b)

            g.wait()
            pltpu.sync_copy(rows, acc_sh.at[dbufs[b]], add=True)
        return carry

    lax.fori_loop(0, NCH // 2, body, 0)
    plsc.subcore_barrier()
    for k in range(RPT // CH):
        pltpu.sync_copy(acc_sh.at[pl.ds(tb + k * CH, CH)], rows)
        pltpu.sync_copy(rows, out_hbm.at[c, pl.ds(tb + k * CH, CH)])


def _sc_scatter(y, src_pad, dst_pad, z_rows):
    k = functools.partial(
        pl.kernel,
        out_type=jax.ShapeDtypeStruct((NC, NP, D), jnp.float32),
        mesh=_sc_mesh(),
        scratch_types=[
            [pltpu.VMEM((CH,), jnp.int32) for _ in range(2)],
            [pltpu.VMEM((CH,), jnp.int32) for _ in range(2)],
            pltpu.VMEM((CH, D), jnp.float32),
            pltpu.SemaphoreType.DMA,
            pltpu.VMEM_SHARED((NP, D), jnp.float32),
        ],
    )(_scatter_body)
    return k(y, src_pad, dst_pad, z_rows)


def _dinv(degp_blk):
    deg = degp_blk[0] + degp_blk[1] + 1.0
    return lax.rsqrt(deg)


def _mm1_body(x_ref, degp_ref, w_ref, o_ref):
    dinv = _dinv(degp_ref[...])
    o_ref[...] = jnp.dot(dinv * x_ref[...], w_ref[...],
                         preferred_element_type=jnp.float32)


def _mm_mid_body(s_ref, y_ref, degp_ref, b_ref, w_ref, o_ref):
    dinv = _dinv(degp_ref[...])
    agg = dinv * (s_ref[0] + s_ref[1] + y_ref[...]) + b_ref[...]
    h = jnp.maximum(agg, 0.0)
    o_ref[...] = jnp.dot(dinv * h, w_ref[...],
                         preferred_element_type=jnp.float32)


def _mm_out_body(s_ref, y_ref, degp_ref, b_ref, w_ref, bl_ref, o_ref):
    dinv = _dinv(degp_ref[...])
    agg = dinv * (s_ref[0] + s_ref[1] + y_ref[...]) + b_ref[...]
    h = jnp.maximum(agg, 0.0)
    o_ref[...] = jnp.dot(h, w_ref[...],
                         preferred_element_type=jnp.float32) + bl_ref[...]


_ROWS = pl.BlockSpec((RB, D), lambda i: (i, 0))
_DEGP = pl.BlockSpec((NC, RB, 1), lambda i: (0, i, 0))
_WMAT = pl.BlockSpec((D, D), lambda i: (0, 0))
_BVEC = pl.BlockSpec((1, D), lambda i: (0, 0))
_SIN = pl.BlockSpec((NC, RB, D), lambda i: (0, i, 0))
_OUT = jax.ShapeDtypeStruct((N, D), jnp.float32)


def _mm1(x, degp3, W):
    return pl.pallas_call(
        _mm1_body, grid=(G,),
        in_specs=[_ROWS, _DEGP, _WMAT],
        out_specs=_ROWS, out_shape=_OUT,
    )(x, degp3, W)


def _mm_mid(s, y, degp3, b, W):
    return pl.pallas_call(
        _mm_mid_body, grid=(G,),
        in_specs=[_SIN, _ROWS, _DEGP, _BVEC, _WMAT],
        out_specs=_ROWS, out_shape=_OUT,
    )(s, y, degp3, b, W)


def _mm_out(s, y, degp3, b, W, bl):
    return pl.pallas_call(
        _mm_out_body, grid=(G,),
        in_specs=[_SIN, _ROWS, _DEGP, _BVEC, _WMAT, _BVEC],
        out_specs=_ROWS, out_shape=_OUT,
    )(s, y, degp3, b, W, bl)


def kernel(x, edge_index, W1, b1, W2, b2, Wl, bl):
    src = edge_index[0]
    dst = edge_index[1]
    pad = EPAD - E
    src_p = jnp.concatenate([src, jnp.zeros((pad,), jnp.int32)])
    dst_p = jnp.concatenate(
        [dst, N + (jnp.arange(pad, dtype=jnp.int32) % (NP - N))])
    z_rows = jnp.zeros((CH, D), jnp.float32)
    b1r = b1.reshape(1, D)
    b2r = b2.reshape(1, D)
    Wlp = jnp.zeros((D, D), jnp.float32).at[:, :C].set(Wl)
    blp = jnp.zeros((1, D), jnp.float32).at[0, :C].set(bl)

    degp = _sc_hist(dst_p)
    degp3 = degp[:, :, None]
    y1 = _mm1(x, degp3, W1)
    s1 = _sc_scatter(y1, src_p, dst_p, z_rows)
    y2 = _mm_mid(s1, y1, degp3, b1r, W2)
    s2 = _sc_scatter(y2, src_p, dst_p, z_rows)
    o = _mm_out(s2, y2, degp3, b2r, Wlp, blp)
    return o[:, :C]


# speedup vs baseline: 1.2186x; 1.2186x over previous
import functools

import jax
import jax.numpy as jnp
from jax import lax
from jax.experimental import pallas as pl
from jax.experimental.pallas import tpu as pltpu
from jax.experimental.pallas import tpu_sc as plsc

N = 10000
E = 320000
D = 128
C = 25

NC = 2
NS = 16
NW = NC * NS
CH = 128
NCH = 79
EPW = CH * NCH
EPAD = EPW * NW
NP = 10240
RPT = NP // NS

RB = 512
G = 20


def _sc_mesh():
    return plsc.VectorSubcoreMesh(core_axis_name="c", subcore_axis_name="s")


def _hist_body(dst_hbm, ones_hbm, z_hbm, out_hbm, dbuf, ones_v, zb, hbuf, hist_sh):
    c = lax.axis_index("c")
    s = lax.axis_index("s")
    tb = s * RPT
    pltpu.sync_copy(ones_hbm, ones_v)
    pltpu.sync_copy(z_hbm, zb)
    for k in range(RPT // CH):
        pltpu.sync_copy(zb, hist_sh.at[pl.ds(tb + k * CH, CH)])
    plsc.subcore_barrier()
    w = c * NS + s

    def body(j, carry):
        base = w * EPW + j * CH
        pltpu.sync_copy(dst_hbm.at[pl.ds(base, CH)], dbuf)
        pltpu.sync_copy(ones_v, hist_sh.at[dbuf], add=True)
        return carry

    lax.fori_loop(0, NCH, body, 0)
    plsc.subcore_barrier()
    pltpu.sync_copy(hist_sh.at[pl.ds(tb, RPT)], hbuf)
    pltpu.sync_copy(hbuf, out_hbm.at[c, pl.ds(tb, RPT)])


def _sc_hist(dst_pad, ones_row, z_row):
    k = functools.partial(
        pl.kernel,
        out_type=jax.ShapeDtypeStruct((NC, NP), jnp.float32),
        mesh=_sc_mesh(),
        scratch_types=[
            pltpu.VMEM((CH,), jnp.int32),
            pltpu.VMEM((CH,), jnp.float32),
            pltpu.VMEM((CH,), jnp.float32),
            pltpu.VMEM((RPT,), jnp.float32),
            pltpu.VMEM_SHARED((NP,), jnp.float32),
        ],
    )(_hist_body)
    return k(dst_pad, ones_row, z_row)


def _scatter_body(y_hbm, src_hbm, dst_hbm, z_hbm, out_hbm,
                  sbuf, dbuf, rows, sem, acc_sh):
    c = lax.axis_index("c")
    s = lax.axis_index("s")
    tb = s * RPT
    pltpu.sync_copy(z_hbm, rows)
    for k in range(RPT // CH):
        pltpu.sync_copy(rows, acc_sh.at[pl.ds(tb + k * CH, CH)])
    plsc.subcore_barrier()
    w = c * NS + s

    def body(j, carry):
        base = w * EPW + j * CH
        pltpu.sync_copy(src_hbm.at[pl.ds(base, CH)], sbuf)
        pltpu.sync_copy(dst_hbm.at[pl.ds(base, CH)], dbuf)
        pltpu.async_copy(y_hbm.at[sbuf], rows, sem).wait()
        pltpu.sync_copy(rows, acc_sh.at[dbuf], add=True)
        return carry

    lax.fori_loop(0, NCH, body, 0)
    plsc.subcore_barrier()
    for k in range(RPT // CH):
        pltpu.sync_copy(acc_sh.at[pl.ds(tb + k * CH, CH)], rows)
        pltpu.sync_copy(rows, out_hbm.at[c, pl.ds(tb + k * CH, CH)])


def _sc_scatter(y, src_pad, dst_pad, z_rows):
    k = functools.partial(
        pl.kernel,
        out_type=jax.ShapeDtypeStruct((NC, NP, D), jnp.float32),
        mesh=_sc_mesh(),
        scratch_types=[
            pltpu.VMEM((CH,), jnp.int32),
            pltpu.VMEM((CH,), jnp.int32),
            pltpu.VMEM((CH, D), jnp.float32),
            pltpu.SemaphoreType.DMA,
            pltpu.VMEM_SHARED((NP, D), jnp.float32),
        ],
    )(_scatter_body)
    return k(y, src_pad, dst_pad, z_rows)


def _dinv(degp_blk):
    deg = degp_blk[0] + degp_blk[1] + 1.0
    return lax.rsqrt(deg)


def _mm1_body(x_ref, degp_ref, w_ref, o_ref):
    dinv = _dinv(degp_ref[...])
    o_ref[...] = jnp.dot(dinv * x_ref[...], w_ref[...],
                         preferred_element_type=jnp.float32)


def _mm_mid_body(s_ref, y_ref, degp_ref, b_ref, w_ref, o_ref):
    dinv = _dinv(degp_ref[...])
    agg = dinv * (s_ref[0] + s_ref[1] + y_ref[...]) + b_ref[...]
    h = jnp.maximum(agg, 0.0)
    o_ref[...] = jnp.dot(dinv * h, w_ref[...],
                         preferred_element_type=jnp.float32)


def _mm_out_body(s_ref, y_ref, degp_ref, b_ref, w_ref, bl_ref, o_ref):
    dinv = _dinv(degp_ref[...])
    agg = dinv * (s_ref[0] + s_ref[1] + y_ref[...]) + b_ref[...]
    h = jnp.maximum(agg, 0.0)
    o_ref[...] = jnp.dot(h, w_ref[...],
                         preferred_element_type=jnp.float32) + bl_ref[...]


_ROWS = pl.BlockSpec((RB, D), lambda i: (i, 0))
_DEGP = pl.BlockSpec((NC, RB, 1), lambda i: (0, i, 0))
_WMAT = pl.BlockSpec((D, D), lambda i: (0, 0))
_BVEC = pl.BlockSpec((1, D), lambda i: (0, 0))
_SIN = pl.BlockSpec((NC, RB, D), lambda i: (0, i, 0))
_OUT = jax.ShapeDtypeStruct((N, D), jnp.float32)


def _mm1(x, degp3, W):
    return pl.pallas_call(
        _mm1_body, grid=(G,),
        in_specs=[_ROWS, _DEGP, _WMAT],
        out_specs=_ROWS, out_shape=_OUT,
    )(x, degp3, W)


def _mm_mid(s, y, degp3, b, W):
    return pl.pallas_call(
        _mm_mid_body, grid=(G,),
        in_specs=[_SIN, _ROWS, _DEGP, _BVEC, _WMAT],
        out_specs=_ROWS, out_shape=_OUT,
    )(s, y, degp3, b, W)


def _mm_out(s, y, degp3, b, W, bl):
    return pl.pallas_call(
        _mm_out_body, grid=(G,),
        in_specs=[_SIN, _ROWS, _DEGP, _BVEC, _WMAT, _BVEC],
        out_specs=_ROWS, out_shape=_OUT,
    )(s, y, degp3, b, W, bl)


def kernel(x, edge_index, W1, b1, W2, b2, Wl, bl):
    src = edge_index[0]
    dst = edge_index[1]
    pad = EPAD - E
    src_p = jnp.concatenate([src, jnp.zeros((pad,), jnp.int32)])
    dst_p = jnp.concatenate([dst, jnp.full((pad,), N, jnp.int32)])
    z_rows = jnp.zeros((CH, D), jnp.float32)
    z_row = jnp.zeros((CH,), jnp.float32)
    ones_row = jnp.ones((CH,), jnp.float32)
    b1r = b1.reshape(1, D)
    b2r = b2.reshape(1, D)
    Wlp = jnp.zeros((D, D), jnp.float32).at[:, :C].set(Wl)
    blp = jnp.zeros((1, D), jnp.float32).at[0, :C].set(bl)

    degp = _sc_hist(dst_p, ones_row, z_row)
    degp3 = degp[:, :, None]
    y1 = _mm1(x, degp3, W1)
    s1 = _sc_scatter(y1, src_p, dst_p, z_rows)
    y2 = _mm_mid(s1, y1, degp3, b1r, W2)
    s2 = _sc_scatter(y2, src_p, dst_p, z_rows)
    o = _mm_out(s2, y2, degp3, b2r, Wlp, blp)
    return o[:, :C]
